# Initial kernel scaffold; baseline (speedup 1.0000x reference)
import functools

import jax
import jax.numpy as jnp
from jax import lax
from jax.experimental import pallas as pl
from jax.experimental.pallas import tpu as pltpu
from jax.experimental.pallas import tpu_sc as plsc

_NC = 2
_NS = 16
_NW = _NC * _NS
_L = 16
_CHUNK = 128


def _sc_aggregate(src_p, dst_p, x, nodes_p, n_acc):
    n_nodes, d = x.shape
    e_pad = src_p.shape[0]
    b_pad = nodes_p.shape[0]
    e_per_w = e_pad // _NW
    n_echunks = e_per_w // _CHUNK
    b_per_s = b_pad // _NS
    n_pchunks = b_per_s // _CHUNK
    b_per_w = b_pad // _NW
    n_xchunks = b_per_w // _CHUNK
    rpt = n_acc // _NS

    zx = jnp.zeros((_CHUNK, d), jnp.float32)
    zd = jnp.zeros((_CHUNK, _L), jnp.float32)
    onecol = zd.at[:, 0].set(1.0)

    mesh = plsc.VectorSubcoreMesh(core_axis_name="c", subcore_axis_name="s")

    @functools.partial(
        pl.kernel,
        out_type=(
            jax.ShapeDtypeStruct((_NC, b_pad, d), jnp.float32),
            jax.ShapeDtypeStruct((_NC, b_pad, _L), jnp.float32),
            jax.ShapeDtypeStruct((b_pad, d), jnp.float32),
        ),
        mesh=mesh,
        scratch_types=[
            pltpu.VMEM_SHARED((n_acc, d), jnp.float32),
            pltpu.VMEM_SHARED((n_acc, _L), jnp.float32),
            pltpu.VMEM((_CHUNK,), jnp.int32),
            pltpu.VMEM((_CHUNK,), jnp.int32),
            pltpu.VMEM((_CHUNK,), jnp.int32),
            pltpu.VMEM((_CHUNK, d), jnp.float32),
            pltpu.VMEM((_CHUNK, _L), jnp.float32),
            pltpu.VMEM((_CHUNK, _L), jnp.float32),
            pltpu.SemaphoreType.DMA,
            pltpu.SemaphoreType.DMA,
        ],
    )
    def k(src_h, dst_h, x_h, nodes_h, zx_h, zd_h, onecol_h,
          psb_h, degb_h, xb_h,
          acc, deg, isrc, idst, indn, rows, ones, dstage, sem, sem2):
        cid = lax.axis_index("c")
        sid = lax.axis_index("s")
        wid = cid * _NS + sid

        pltpu.sync_copy(zx_h, rows)
        pltpu.sync_copy(zd_h, dstage)
        base = sid * rpt
        off = 0
        for _ in range(rpt // _CHUNK):
            pltpu.sync_copy(rows, acc.at[pl.ds(base + off, _CHUNK)])
            pltpu.sync_copy(dstage, deg.at[pl.ds(base + off, _CHUNK)])
            off += _CHUNK
        rem = rpt - off
        if rem:
            pltpu.sync_copy(rows.at[pl.ds(0, rem)], acc.at[pl.ds(base + off, rem)])
            pltpu.sync_copy(dstage.at[pl.ds(0, rem)], deg.at[pl.ds(base + off, rem)])
        pltpu.sync_copy(onecol_h, ones)
        plsc.subcore_barrier()

        ebase = wid * e_per_w

        @pl.loop(0, n_echunks)
        def _edges(i):
            e0 = ebase + i * _CHUNK
            pltpu.sync_copy(src_h.at[pl.ds(e0, _CHUNK)], isrc)
            pltpu.sync_copy(dst_h.at[pl.ds(e0, _CHUNK)], idst)
            pltpu.async_copy(x_h.at[isrc], rows, sem).wait()
            pltpu.sync_copy(rows, acc.at[idst], add=True)
            pltpu.sync_copy(ones, deg.at[idst], add=True)

        plsc.subcore_barrier()

        @pl.loop(0, n_pchunks)
        def _psb(k_):
            nb = sid * b_per_s + k_ * _CHUNK
            pltpu.sync_copy(nodes_h.at[pl.ds(nb, _CHUNK)], indn)
            pltpu.async_copy(acc.at[indn], rows, sem).wait()
            pltpu.sync_copy(rows, psb_h.at[cid, pl.ds(nb, _CHUNK)])
            pltpu.async_copy(deg.at[indn], dstage, sem2).wait()
            pltpu.sync_copy(dstage, degb_h.at[cid, pl.ds(nb, _CHUNK)])

        @pl.loop(0, n_xchunks)
        def _xb(k_):
            nb = wid * b_per_w + k_ * _CHUNK
            pltpu.sync_copy(nodes_h.at[pl.ds(nb, _CHUNK)], indn)
            pltpu.async_copy(x_h.at[indn], rows, sem).wait()
            pltpu.sync_copy(rows, xb_h.at[pl.ds(nb, _CHUNK)])

    return k(src_p, dst_p, x, nodes_p, zx, zd, onecol)


def _tc_body(xb_ref, p0_ref, p1_ref, d0_ref, d1_ref, we_ref, wo_ref, b_ref,
             o_ref):
    deg = jnp.sum(d0_ref[...] + d1_ref[...], axis=1, keepdims=True)
    nm = (p0_ref[...] + p1_ref[...]) / jnp.maximum(deg, 1.0)
    cat = jnp.concatenate([xb_ref[...], nm], axis=1)
    h = lax.dot_general(cat, we_ref[...], (((1,), (1,)), ((), ())),
                        preferred_element_type=jnp.float32)
    h = jnp.maximum(h, 0.0)
    o = lax.dot_general(h, wo_ref[...], (((1,), (1,)), ((), ())),
                        preferred_element_type=jnp.float32)
    o_ref[...] = o + b_ref[...]


def kernel(nodes, x, edge_index, W_enc, W_out, b_out):
    n_nodes, d = x.shape
    n_edges = edge_index.shape[1]
    batch = nodes.shape[0]
    d_out = W_out.shape[0]

    grain = _NW * _CHUNK
    e_pad = -(-n_edges // grain) * grain
    src_p = jnp.concatenate(
        [edge_index[0], jnp.zeros((e_pad - n_edges,), jnp.int32)])
    dst_p = jnp.concatenate(
        [edge_index[1], jnp.full((e_pad - n_edges,), n_nodes, jnp.int32)])
    n_acc = -(-(n_nodes + 1) // _NS) * _NS
    b_pad = -(-batch // grain) * grain
    nodes_p = jnp.concatenate(
        [nodes, jnp.zeros((b_pad - batch,), jnp.int32)])

    psb, degb, xb = _sc_aggregate(src_p, dst_p, x, nodes_p, n_acc)

    bb = min(1024, b_pad)
    out = pl.pallas_call(
        _tc_body,
        grid=(b_pad // bb,),
        in_specs=[
            pl.BlockSpec((bb, d), lambda i: (i, 0)),
            pl.BlockSpec((bb, d), lambda i: (i, 0)),
            pl.BlockSpec((bb, d), lambda i: (i, 0)),
            pl.BlockSpec((bb, _L), lambda i: (i, 0)),
            pl.BlockSpec((bb, _L), lambda i: (i, 0)),
            pl.BlockSpec((W_enc.shape[0], 2 * d), lambda i: (0, 0)),
            pl.BlockSpec((d_out, W_out.shape[1]), lambda i: (0, 0)),
            pl.BlockSpec((1, d_out), lambda i: (0, 0)),
        ],
        out_specs=pl.BlockSpec((bb, d_out), lambda i: (i, 0)),
        out_shape=jax.ShapeDtypeStruct((b_pad, d_out), jnp.float32),
    )(xb, psb[0], psb[1], degb[0], degb[1], W_enc, W_out,
      b_out.reshape(1, d_out))
    return out[:batch]


# speedup vs baseline: 4.4271x; 4.4271x over previous
import functools

import jax
import jax.numpy as jnp
from jax import lax
from jax.experimental import pallas as pl
from jax.experimental.pallas import tpu as pltpu
from jax.experimental.pallas import tpu_sc as plsc

_NC = 2
_NS = 16
_NW = _NC * _NS
_L = 16
_CHUNK = 128


def _sc_aggregate(src_p, dst_p, x, nodes_p, n_acc):
    n_nodes, d = x.shape
    e_pad = src_p.shape[0]
    b_pad = nodes_p.shape[0]
    e_per_w = e_pad // _NW
    n_echunks = e_per_w // _CHUNK
    b_per_s = b_pad // _NS
    n_pchunks = b_per_s // _CHUNK
    b_per_w = b_pad // _NW
    n_xchunks = b_per_w // _CHUNK
    rpt = n_acc // _NS

    mesh = plsc.VectorSubcoreMesh(core_axis_name="c", subcore_axis_name="s")

    @functools.partial(
        pl.kernel,
        out_type=(
            jax.ShapeDtypeStruct((_NC, b_pad, d), jnp.float32),
            jax.ShapeDtypeStruct((b_pad, d), jnp.float32),
        ),
        mesh=mesh,
        scratch_types=[
            pltpu.VMEM_SHARED((n_acc, d), jnp.float32),
            pltpu.VMEM((_CHUNK,), jnp.int32),
            pltpu.VMEM((_CHUNK,), jnp.int32),
            pltpu.VMEM((_CHUNK, d), jnp.float32),
            pltpu.SemaphoreType.DMA,
        ],
    )
    def k(src_h, dst_h, x_h, nodes_h, psb_h, xb_h,
          acc, isrc, idst, rows, sem):
        cid = lax.axis_index("c")
        sid = lax.axis_index("s")
        wid = cid * _NS + sid

        zv = jnp.zeros((_L,), jnp.float32)

        @pl.loop(0, _CHUNK)
        def _zrow(r):
            for c in range(d // _L):
                rows[r, pl.ds(c * _L, _L)] = zv

        base = sid * rpt
        off = 0
        for _ in range(rpt // _CHUNK):
            pltpu.sync_copy(rows, acc.at[pl.ds(base + off, _CHUNK)])
            off += _CHUNK
        rem = rpt - off
        if rem:
            pltpu.sync_copy(rows.at[pl.ds(0, rem)], acc.at[pl.ds(base + off, rem)])
        plsc.subcore_barrier()

        ebase = wid * e_per_w

        @pl.loop(0, n_echunks)
        def _edges(i):
            e0 = ebase + i * _CHUNK
            pltpu.sync_copy(src_h.at[pl.ds(e0, _CHUNK)], isrc)
            pltpu.sync_copy(dst_h.at[pl.ds(e0, _CHUNK)], idst)
            pltpu.async_copy(x_h.at[isrc], rows, sem).wait()
            pltpu.sync_copy(rows, acc.at[idst], add=True)

        plsc.subcore_barrier()

        @pl.loop(0, n_pchunks)
        def _psb(k_):
            nb = sid * b_per_s + k_ * _CHUNK
            pltpu.sync_copy(nodes_h.at[pl.ds(nb, _CHUNK)], idst)
            pltpu.async_copy(acc.at[idst], rows, sem).wait()
            pltpu.sync_copy(rows, psb_h.at[cid, pl.ds(nb, _CHUNK)])

        @pl.loop(0, n_xchunks)
        def _xb(k_):
            nb = wid * b_per_w + k_ * _CHUNK
            pltpu.sync_copy(nodes_h.at[pl.ds(nb, _CHUNK)], idst)
            pltpu.async_copy(x_h.at[idst], rows, sem).wait()
            pltpu.sync_copy(rows, xb_h.at[pl.ds(nb, _CHUNK)])

    return k(src_p, dst_p, x, nodes_p)


def _sc_degree(dst_p, nodes_p, n_acc, d):
    e_pad = dst_p.shape[0]
    b_pad = nodes_p.shape[0]
    e_per_w = e_pad // _NW
    n_echunks = e_per_w // _CHUNK
    b_per_s = b_pad // _NS
    n_pchunks = b_per_s // _CHUNK
    rpt = n_acc // _NS

    mesh = plsc.VectorSubcoreMesh(core_axis_name="c", subcore_axis_name="s")

    @functools.partial(
        pl.kernel,
        out_type=jax.ShapeDtypeStruct((_NC, b_pad, d), jnp.float32),
        mesh=mesh,
        scratch_types=[
            pltpu.VMEM_SHARED((n_acc, d), jnp.float32),
            pltpu.VMEM((_CHUNK,), jnp.int32),
            pltpu.VMEM((_CHUNK, d), jnp.float32),
            pltpu.SemaphoreType.DMA,
        ],
    )
    def k(dst_h, nodes_h, degb_h, deg, idst, ones, sem):
        cid = lax.axis_index("c")
        sid = lax.axis_index("s")
        wid = cid * _NS + sid

        zv = jnp.zeros((_L,), jnp.float32)

        @pl.loop(0, _CHUNK)
        def _zrow(r):
            for c in range(d // _L):
                ones[r, pl.ds(c * _L, _L)] = zv

        base = sid * rpt
        off = 0
        for _ in range(rpt // _CHUNK):
            pltpu.sync_copy(ones, deg.at[pl.ds(base + off, _CHUNK)])
            off += _CHUNK
        rem = rpt - off
        if rem:
            pltpu.sync_copy(ones.at[pl.ds(0, rem)], deg.at[pl.ds(base + off, rem)])

        hot = jnp.where(lax.iota(jnp.int32, _L) == 0, 1.0, 0.0)

        @pl.loop(0, _CHUNK)
        def _hotrow(r):
            ones[r, pl.ds(0, _L)] = hot

        plsc.subcore_barrier()

        ebase = wid * e_per_w

        @pl.loop(0, n_echunks)
        def _edges(i):
            pltpu.sync_copy(dst_h.at[pl.ds(ebase + i * _CHUNK, _CHUNK)], idst)
            pltpu.sync_copy(ones, deg.at[idst], add=True)

        plsc.subcore_barrier()

        @pl.loop(0, n_pchunks)
        def _degb(k_):
            nb = sid * b_per_s + k_ * _CHUNK
            pltpu.sync_copy(nodes_h.at[pl.ds(nb, _CHUNK)], idst)
            pltpu.async_copy(deg.at[idst], ones, sem).wait()
            pltpu.sync_copy(ones, degb_h.at[cid, pl.ds(nb, _CHUNK)])

    return k(dst_p, nodes_p)


def _tc_body(xb_ref, p0_ref, p1_ref, d0_ref, d1_ref, we_ref, wo_ref, b_ref,
             o_ref):
    deg = jnp.sum(d0_ref[...] + d1_ref[...], axis=1, keepdims=True)
    nm = (p0_ref[...] + p1_ref[...]) / jnp.maximum(deg, 1.0)
    cat = jnp.concatenate([xb_ref[...], nm], axis=1)
    h = lax.dot_general(cat, we_ref[...], (((1,), (1,)), ((), ())),
                        preferred_element_type=jnp.float32)
    h = jnp.maximum(h, 0.0)
    o = lax.dot_general(h, wo_ref[...], (((1,), (1,)), ((), ())),
                        preferred_element_type=jnp.float32)
    o_ref[...] = o + b_ref[...]


def kernel(nodes, x, edge_index, W_enc, W_out, b_out):
    n_nodes, d = x.shape
    n_edges = edge_index.shape[1]
    batch = nodes.shape[0]
    d_out = W_out.shape[0]

    grain = _NW * _CHUNK
    e_pad = -(-n_edges // grain) * grain
    src_p = jnp.concatenate(
        [edge_index[0], jnp.zeros((e_pad - n_edges,), jnp.int32)])
    dst_p = jnp.concatenate(
        [edge_index[1], jnp.full((e_pad - n_edges,), n_nodes, jnp.int32)])
    n_acc = -(-(n_nodes + 1) // (_NS * 8)) * (_NS * 8)
    b_pad = -(-batch // grain) * grain
    nodes_p = jnp.concatenate(
        [nodes, jnp.zeros((b_pad - batch,), jnp.int32)])

    psb, xb = _sc_aggregate(src_p, dst_p, x, nodes_p, n_acc)
    degb = _sc_degree(dst_p, nodes_p, n_acc, d)

    bb = min(1024, b_pad)
    out = pl.pallas_call(
        _tc_body,
        grid=(b_pad // bb,),
        in_specs=[
            pl.BlockSpec((bb, d), lambda i: (i, 0)),
            pl.BlockSpec((bb, d), lambda i: (i, 0)),
            pl.BlockSpec((bb, d), lambda i: (i, 0)),
            pl.BlockSpec((bb, d), lambda i: (i, 0)),
            pl.BlockSpec((bb, d), lambda i: (i, 0)),
            pl.BlockSpec((W_enc.shape[0], 2 * d), lambda i: (0, 0)),
            pl.BlockSpec((d_out, W_out.shape[1]), lambda i: (0, 0)),
            pl.BlockSpec((1, d_out), lambda i: (0, 0)),
        ],
        out_specs=pl.BlockSpec((bb, d_out), lambda i: (i, 0)),
        out_shape=jax.ShapeDtypeStruct((b_pad, d_out), jnp.float32),
    )(xb, psb[0], psb[1], degb[0], degb[1], W_enc, W_out,
      b_out.reshape(1, d_out))
    return out[:batch]


# speedup vs baseline: 4.9355x; 1.1148x over previous
import functools

import jax
import jax.numpy as jnp
from jax import lax
from jax.experimental import pallas as pl
from jax.experimental.pallas import tpu as pltpu
from jax.experimental.pallas import tpu_sc as plsc

_NC = 2
_NS = 16
_NW = _NC * _NS
_L = 16
_CHUNK = 128


def _sc_aggregate(src_p, dst_p, x, nodes_p, n_acc):
    n_nodes, d = x.shape
    e_pad = src_p.shape[0]
    b_pad = nodes_p.shape[0]
    e_per_w = e_pad // _NW
    n_echunks = e_per_w // _CHUNK
    b_per_s = b_pad // _NS
    n_pchunks = b_per_s // _CHUNK
    b_per_w = b_pad // _NW
    n_xchunks = b_per_w // _CHUNK
    rpt = n_acc // _NS

    mesh = plsc.VectorSubcoreMesh(core_axis_name="c", subcore_axis_name="s")

    @functools.partial(
        pl.kernel,
        out_type=(
            jax.ShapeDtypeStruct((_NC, b_pad, d), jnp.float32),
            jax.ShapeDtypeStruct((b_pad, d), jnp.float32),
        ),
        mesh=mesh,
        scratch_types=[
            pltpu.VMEM_SHARED((n_acc, d), jnp.float32),
            pltpu.VMEM((_CHUNK,), jnp.int32),
            pltpu.VMEM((_CHUNK,), jnp.int32),
            pltpu.VMEM((_CHUNK, d), jnp.float32),
            pltpu.SemaphoreType.DMA,
        ],
    )
    def k(src_h, dst_h, x_h, nodes_h, psb_h, xb_h,
          acc, isrc, idst, rows, sem):
        cid = lax.axis_index("c")
        sid = lax.axis_index("s")
        wid = cid * _NS + sid

        zv = jnp.zeros((_L,), jnp.float32)

        @pl.loop(0, _CHUNK)
        def _zrow(r):
            for c in range(d // _L):
                rows[r, pl.ds(c * _L, _L)] = zv

        base = sid * rpt
        off = 0
        for _ in range(rpt // _CHUNK):
            pltpu.sync_copy(rows, acc.at[pl.ds(base + off, _CHUNK)])
            off += _CHUNK
        rem = rpt - off
        if rem:
            pltpu.sync_copy(rows.at[pl.ds(0, rem)], acc.at[pl.ds(base + off, rem)])
        plsc.subcore_barrier()

        ebase = wid * e_per_w

        @pl.loop(0, n_echunks)
        def _edges(i):
            e0 = ebase + i * _CHUNK
            pltpu.sync_copy(src_h.at[pl.ds(e0, _CHUNK)], isrc)
            pltpu.sync_copy(dst_h.at[pl.ds(e0, _CHUNK)], idst)
            pltpu.async_copy(x_h.at[isrc], rows, sem).wait()
            pltpu.sync_copy(rows, acc.at[idst], add=True)

        plsc.subcore_barrier()

        @pl.loop(0, n_pchunks)
        def _psb(k_):
            nb = sid * b_per_s + k_ * _CHUNK
            pltpu.sync_copy(nodes_h.at[pl.ds(nb, _CHUNK)], idst)
            pltpu.async_copy(acc.at[idst], rows, sem).wait()
            pltpu.sync_copy(rows, psb_h.at[cid, pl.ds(nb, _CHUNK)])

        @pl.loop(0, n_xchunks)
        def _xb(k_):
            nb = wid * b_per_w + k_ * _CHUNK
            pltpu.sync_copy(nodes_h.at[pl.ds(nb, _CHUNK)], idst)
            pltpu.async_copy(x_h.at[idst], rows, sem).wait()
            pltpu.sync_copy(rows, xb_h.at[pl.ds(nb, _CHUNK)])

    return k(src_p, dst_p, x, nodes_p)


def _sc_degree(dst_p, nodes_p, n_acc):
    e_pad = dst_p.shape[0]
    b_pad = nodes_p.shape[0]
    e_per_w = e_pad // _NW
    n_echunks = e_per_w // _CHUNK
    b_per_s = b_pad // _NS
    n_pchunks = b_per_s // _CHUNK
    rpt = n_acc // _NS

    mesh = plsc.VectorSubcoreMesh(core_axis_name="c", subcore_axis_name="s")

    @functools.partial(
        pl.kernel,
        out_type=jax.ShapeDtypeStruct((_NC, b_pad), jnp.float32),
        mesh=mesh,
        scratch_types=[
            pltpu.VMEM_SHARED((n_acc,), jnp.float32),
            pltpu.VMEM((_CHUNK,), jnp.int32),
            pltpu.VMEM((_CHUNK,), jnp.float32),
            pltpu.SemaphoreType.DMA,
        ],
    )
    def k(dst_h, nodes_h, degb_h, deg, idst, ones, sem):
        cid = lax.axis_index("c")
        sid = lax.axis_index("s")
        wid = cid * _NS + sid

        zv = jnp.zeros((_L,), jnp.float32)
        for c in range(_CHUNK // _L):
            ones[pl.ds(c * _L, _L)] = zv

        base = sid * rpt
        off = 0
        for _ in range(rpt // _CHUNK):
            pltpu.sync_copy(ones, deg.at[pl.ds(base + off, _CHUNK)])
            off += _CHUNK
        rem = rpt - off
        if rem:
            pltpu.sync_copy(ones.at[pl.ds(0, rem)], deg.at[pl.ds(base + off, rem)])

        ov = jnp.ones((_L,), jnp.float32)
        for c in range(_CHUNK // _L):
            ones[pl.ds(c * _L, _L)] = ov

        plsc.subcore_barrier()

        ebase = wid * e_per_w

        @pl.loop(0, n_echunks)
        def _edges(i):
            pltpu.sync_copy(dst_h.at[pl.ds(ebase + i * _CHUNK, _CHUNK)], idst)
            pltpu.sync_copy(ones, deg.at[idst], add=True)

        plsc.subcore_barrier()

        @pl.loop(0, n_pchunks)
        def _degb(k_):
            nb = sid * b_per_s + k_ * _CHUNK
            pltpu.sync_copy(nodes_h.at[pl.ds(nb, _CHUNK)], idst)
            pltpu.async_copy(deg.at[idst], ones, sem).wait()
            pltpu.sync_copy(ones, degb_h.at[cid, pl.ds(nb, _CHUNK)])

    return k(dst_p, nodes_p)


def _tc_body(xb_ref, p0_ref, p1_ref, d0_ref, d1_ref, we_ref, wo_ref, b_ref,
             o_ref):
    deg = d0_ref[...] + d1_ref[...]
    nm = (p0_ref[...] + p1_ref[...]) / jnp.maximum(deg, 1.0)
    cat = jnp.concatenate([xb_ref[...], nm], axis=1)
    h = lax.dot_general(cat, we_ref[...], (((1,), (1,)), ((), ())),
                        preferred_element_type=jnp.float32)
    h = jnp.maximum(h, 0.0)
    o = lax.dot_general(h, wo_ref[...], (((1,), (1,)), ((), ())),
                        preferred_element_type=jnp.float32)
    o_ref[...] = o + b_ref[...]


def kernel(nodes, x, edge_index, W_enc, W_out, b_out):
    n_nodes, d = x.shape
    n_edges = edge_index.shape[1]
    batch = nodes.shape[0]
    d_out = W_out.shape[0]

    grain = _NW * _CHUNK
    e_pad = -(-n_edges // grain) * grain
    src_p = jnp.concatenate(
        [edge_index[0], jnp.zeros((e_pad - n_edges,), jnp.int32)])
    dst_p = jnp.concatenate(
        [edge_index[1], jnp.full((e_pad - n_edges,), n_nodes, jnp.int32)])
    n_acc = -(-(n_nodes + 1) // (_NS * 8)) * (_NS * 8)
    b_pad = -(-batch // grain) * grain
    nodes_p = jnp.concatenate(
        [nodes, jnp.zeros((b_pad - batch,), jnp.int32)])

    psb, xb = _sc_aggregate(src_p, dst_p, x, nodes_p, n_acc)
    degb = _sc_degree(dst_p, nodes_p, n_acc)

    bb = min(1024, b_pad)
    out = pl.pallas_call(
        _tc_body,
        grid=(b_pad // bb,),
        in_specs=[
            pl.BlockSpec((bb, d), lambda i: (i, 0)),
            pl.BlockSpec((bb, d), lambda i: (i, 0)),
            pl.BlockSpec((bb, d), lambda i: (i, 0)),
            pl.BlockSpec((bb, 1), lambda i: (i, 0)),
            pl.BlockSpec((bb, 1), lambda i: (i, 0)),
            pl.BlockSpec((W_enc.shape[0], 2 * d), lambda i: (0, 0)),
            pl.BlockSpec((d_out, W_out.shape[1]), lambda i: (0, 0)),
            pl.BlockSpec((1, d_out), lambda i: (0, 0)),
        ],
        out_specs=pl.BlockSpec((bb, d_out), lambda i: (i, 0)),
        out_shape=jax.ShapeDtypeStruct((b_pad, d_out), jnp.float32),
    )(xb, psb[0], psb[1], degb[0].reshape(b_pad, 1), degb[1].reshape(b_pad, 1),
      W_enc, W_out, b_out.reshape(1, d_out))
    return out[:batch]


# speedup vs baseline: 5.3740x; 1.0888x over previous
import functools

import jax
import jax.numpy as jnp
from jax import lax
from jax.experimental import pallas as pl
from jax.experimental.pallas import tpu as pltpu
from jax.experimental.pallas import tpu_sc as plsc

_NC = 2
_NS = 16
_NW = _NC * _NS
_L = 16
_CHUNK = 128


def _sc_aggregate(src_p, dst_p, x, nodes_p, n_acc):
    n_nodes, d = x.shape
    e_pad = src_p.shape[0]
    b_pad = nodes_p.shape[0]
    e_per_w = e_pad // _NW
    n_echunks = e_per_w // _CHUNK
    b_per_s = b_pad // _NS
    n_pchunks = b_per_s // _CHUNK
    b_per_w = b_pad // _NW
    n_xchunks = b_per_w // _CHUNK
    rpt = n_acc // _NS

    mesh = plsc.VectorSubcoreMesh(core_axis_name="c", subcore_axis_name="s")

    @functools.partial(
        pl.kernel,
        out_type=(
            jax.ShapeDtypeStruct((_NC, b_pad, d), jnp.float32),
            jax.ShapeDtypeStruct((b_pad, d), jnp.float32),
            jax.ShapeDtypeStruct((_NC, b_pad), jnp.float32),
        ),
        mesh=mesh,
        scratch_types=[
            pltpu.VMEM_SHARED((n_acc, d), jnp.float32),
            pltpu.VMEM_SHARED((n_acc,), jnp.float32),
            pltpu.VMEM((_CHUNK,), jnp.int32),
            pltpu.VMEM((_CHUNK,), jnp.int32),
            pltpu.VMEM((_CHUNK, d), jnp.float32),
            pltpu.VMEM((_CHUNK,), jnp.float32),
            pltpu.SemaphoreType.DMA,
        ],
    )
    def k(src_h, dst_h, x_h, nodes_h, psb_h, xb_h, degb_h,
          acc, deg, isrc, idst, rows, ones, sem):
        cid = lax.axis_index("c")
        sid = lax.axis_index("s")
        wid = cid * _NS + sid

        zv = jnp.zeros((_L,), jnp.float32)

        @pl.loop(0, _CHUNK)
        def _zrow(r):
            for c in range(d // _L):
                rows[r, pl.ds(c * _L, _L)] = zv

        for c in range(_CHUNK // _L):
            ones[pl.ds(c * _L, _L)] = zv

        base = sid * rpt
        off = 0
        for _ in range(rpt // _CHUNK):
            pltpu.sync_copy(rows, acc.at[pl.ds(base + off, _CHUNK)])
            pltpu.sync_copy(ones, deg.at[pl.ds(base + off, _CHUNK)])
            off += _CHUNK
        rem = rpt - off
        if rem:
            pltpu.sync_copy(rows.at[pl.ds(0, rem)], acc.at[pl.ds(base + off, rem)])
            pltpu.sync_copy(ones.at[pl.ds(0, rem)], deg.at[pl.ds(base + off, rem)])

        ov = jnp.ones((_L,), jnp.float32)
        for c in range(_CHUNK // _L):
            ones[pl.ds(c * _L, _L)] = ov

        plsc.subcore_barrier()

        ebase = wid * e_per_w

        @pl.loop(0, n_echunks)
        def _edges(i):
            e0 = ebase + i * _CHUNK
            pltpu.sync_copy(src_h.at[pl.ds(e0, _CHUNK)], isrc)
            pltpu.sync_copy(dst_h.at[pl.ds(e0, _CHUNK)], idst)
            pltpu.async_copy(x_h.at[isrc], rows, sem).wait()
            pltpu.sync_copy(rows, acc.at[idst], add=True)
            pltpu.sync_copy(ones, deg.at[idst], add=True)

        plsc.subcore_barrier()

        @pl.loop(0, n_pchunks)
        def _psb(k_):
            nb = sid * b_per_s + k_ * _CHUNK
            pltpu.sync_copy(nodes_h.at[pl.ds(nb, _CHUNK)], idst)
            pltpu.async_copy(acc.at[idst], rows, sem).wait()
            pltpu.sync_copy(rows, psb_h.at[cid, pl.ds(nb, _CHUNK)])
            pltpu.async_copy(deg.at[idst], ones, sem).wait()
            pltpu.sync_copy(ones, degb_h.at[cid, pl.ds(nb, _CHUNK)])

        @pl.loop(0, n_xchunks)
        def _xb(k_):
            nb = wid * b_per_w + k_ * _CHUNK
            pltpu.sync_copy(nodes_h.at[pl.ds(nb, _CHUNK)], idst)
            pltpu.async_copy(x_h.at[idst], rows, sem).wait()
            pltpu.sync_copy(rows, xb_h.at[pl.ds(nb, _CHUNK)])

    return k(src_p, dst_p, x, nodes_p)


def _tc_body(xb_ref, p0_ref, p1_ref, d0_ref, d1_ref, we_ref, wo_ref, b_ref,
             o_ref):
    deg = d0_ref[...] + d1_ref[...]
    nm = (p0_ref[...] + p1_ref[...]) / jnp.maximum(deg, 1.0)
    cat = jnp.concatenate([xb_ref[...], nm], axis=1)
    h = lax.dot_general(cat, we_ref[...], (((1,), (1,)), ((), ())),
                        preferred_element_type=jnp.float32)
    h = jnp.maximum(h, 0.0)
    o = lax.dot_general(h, wo_ref[...], (((1,), (1,)), ((), ())),
                        preferred_element_type=jnp.float32)
    o_ref[...] = o + b_ref[...]


def kernel(nodes, x, edge_index, W_enc, W_out, b_out):
    n_nodes, d = x.shape
    n_edges = edge_index.shape[1]
    batch = nodes.shape[0]
    d_out = W_out.shape[0]

    grain = _NW * _CHUNK
    e_pad = -(-n_edges // grain) * grain
    src_p = jnp.concatenate(
        [edge_index[0], jnp.zeros((e_pad - n_edges,), jnp.int32)])
    dst_p = jnp.concatenate(
        [edge_index[1], jnp.full((e_pad - n_edges,), n_nodes, jnp.int32)])
    n_acc = -(-(n_nodes + 1) // (_NS * 8)) * (_NS * 8)
    b_pad = -(-batch // grain) * grain
    nodes_p = jnp.concatenate(
        [nodes, jnp.zeros((b_pad - batch,), jnp.int32)])

    psb, xb, degb = _sc_aggregate(src_p, dst_p, x, nodes_p, n_acc)

    bb = min(1024, b_pad)
    out = pl.pallas_call(
        _tc_body,
        grid=(b_pad // bb,),
        in_specs=[
            pl.BlockSpec((bb, d), lambda i: (i, 0)),
            pl.BlockSpec((bb, d), lambda i: (i, 0)),
            pl.BlockSpec((bb, d), lambda i: (i, 0)),
            pl.BlockSpec((bb, 1), lambda i: (i, 0)),
            pl.BlockSpec((bb, 1), lambda i: (i, 0)),
            pl.BlockSpec((W_enc.shape[0], 2 * d), lambda i: (0, 0)),
            pl.BlockSpec((d_out, W_out.shape[1]), lambda i: (0, 0)),
            pl.BlockSpec((1, d_out), lambda i: (0, 0)),
        ],
        out_specs=pl.BlockSpec((bb, d_out), lambda i: (i, 0)),
        out_shape=jax.ShapeDtypeStruct((b_pad, d_out), jnp.float32),
    )(xb, psb[0], psb[1], degb[0].reshape(b_pad, 1), degb[1].reshape(b_pad, 1),
      W_enc, W_out, b_out.reshape(1, d_out))
    return out[:batch]
